# Initial kernel scaffold; baseline (speedup 1.0000x reference)
import jax
import jax.numpy as jnp
from jax.experimental import pallas as pl


def kernel(sent_ids, learn_embed, gate_W, gate_b, lin_W, lin_b, nonlin_W, nonlin_b):
    raise NotImplementedError("write your pallas kernel here")


# speedup vs baseline: 1.2594x; 1.2594x over previous
import functools

import jax
import jax.numpy as jnp
from jax import lax
from jax.experimental import pallas as pl
from jax.experimental.pallas import tpu as pltpu
from jax.experimental.pallas import tpu_sc as plsc

_B, _L, _V, _LDIM, _DIM = 16384, 50, 1000000, 64, 128
_N = _B * _L

_NC, _NS = 2, 16
_NW = _NC * _NS
_ROWS_PER_W = _N // _NW
_IDX_MINOR = 128
_DMAS_PER_CHUNK = 8
_CHUNK = _DMAS_PER_CHUNK * _IDX_MINOR
_NCH = _ROWS_PER_W // _CHUNK


def _sc_gather(table, ids4d):
    mesh = plsc.VectorSubcoreMesh(core_axis_name="c", subcore_axis_name="s")

    @functools.partial(
        pl.kernel,
        mesh=mesh,
        out_type=jax.ShapeDtypeStruct(
            (_NW, _NCH, _DMAS_PER_CHUNK, _IDX_MINOR, _LDIM), jnp.float32),
        scratch_types=[
            pltpu.VMEM((_DMAS_PER_CHUNK, _IDX_MINOR), jnp.int32),
            pltpu.VMEM((_DMAS_PER_CHUNK, _IDX_MINOR, _LDIM), jnp.float32),
            pltpu.SemaphoreType.DMA,
        ],
        compiler_params=pltpu.CompilerParams(use_tc_tiling_on_sc=False),
    )
    def k(table_hbm, ids_hbm, out_hbm, idx_v, rows_v, sem):
        wid = lax.axis_index("s") * _NC + lax.axis_index("c")

        def body(ch, carry):
            pltpu.sync_copy(ids_hbm.at[wid, ch], idx_v)
            descs = []
            for j in range(_DMAS_PER_CHUNK):
                descs.append(pltpu.async_copy(
                    table_hbm.at[idx_v.at[j]], rows_v.at[j], sem))
            for d in descs:
                d.wait()
            pltpu.sync_copy(rows_v, out_hbm.at[wid, ch])
            return carry

        lax.fori_loop(0, _NCH, body, 0)

    return k(table, ids4d)


def _tc_highway(pre, w_cat, b_cat):
    rows = 2048
    grid = (_N // rows,)

    def body(x_ref, w_ref, b_ref, o_ref):
        x = x_ref[...]
        h = jnp.dot(x, w_ref[...], preferred_element_type=jnp.float32)
        h = h + b_ref[...]
        gate = 1.0 / (1.0 + jnp.exp(-h[:, :_DIM]))
        lin = h[:, _DIM:2 * _DIM]
        nonlin = jnp.maximum(h[:, 2 * _DIM:], 0.0)
        o_ref[...] = gate * (nonlin - lin) + lin

    return pl.pallas_call(
        body,
        grid=grid,
        in_specs=[
            pl.BlockSpec((rows, _LDIM), lambda i: (i, 0)),
            pl.BlockSpec((_LDIM, 3 * _DIM), lambda i: (0, 0)),
            pl.BlockSpec((1, 3 * _DIM), lambda i: (0, 0)),
        ],
        out_specs=pl.BlockSpec((rows, _DIM), lambda i: (i, 0)),
        out_shape=jax.ShapeDtypeStruct((_N, _DIM), jnp.float32),
    )(pre, w_cat, b_cat)


def kernel(sent_ids, learn_embed, gate_W, gate_b, lin_W, lin_b, nonlin_W, nonlin_b):
    ids = sent_ids.reshape(_NW, _NCH, _DMAS_PER_CHUNK, _IDX_MINOR).astype(jnp.int32)
    pre = _sc_gather(learn_embed, ids).reshape(_N, _LDIM)
    w_cat = jnp.concatenate([gate_W, lin_W, nonlin_W], axis=1)
    b_cat = jnp.concatenate([gate_b - 2.0, lin_b, nonlin_b]).reshape(1, 3 * _DIM)
    out = _tc_highway(pre, w_cat, b_cat)
    return out.reshape(_B, _L, _DIM)


# speedup vs baseline: 2.2254x; 1.7671x over previous
import functools

import jax
import jax.numpy as jnp
from jax import lax
from jax.experimental import pallas as pl
from jax.experimental.pallas import tpu as pltpu
from jax.experimental.pallas import tpu_sc as plsc

_B, _L, _V, _LDIM, _DIM = 16384, 50, 1000000, 64, 128
_N = _B * _L

_NC, _NS = 2, 16
_NW = _NC * _NS
_ROWS_PER_W = _N // _NW
_IDX_MINOR = 128
_DMAS_PER_CHUNK = 8
_CHUNK = _DMAS_PER_CHUNK * _IDX_MINOR
_NCH = _ROWS_PER_W // _CHUNK


def _sc_gather(table, ids4d):
    mesh = plsc.VectorSubcoreMesh(core_axis_name="c", subcore_axis_name="s")

    @functools.partial(
        pl.kernel,
        mesh=mesh,
        out_type=jax.ShapeDtypeStruct(
            (_NW, _NCH, _DMAS_PER_CHUNK, _IDX_MINOR, _LDIM), jnp.float32),
        scratch_types=[
            pltpu.VMEM((_DMAS_PER_CHUNK, _IDX_MINOR), jnp.int32),
            pltpu.VMEM((_DMAS_PER_CHUNK, _IDX_MINOR, _LDIM), jnp.float32),
            pltpu.SemaphoreType.DMA,
        ],
        compiler_params=pltpu.CompilerParams(use_tc_tiling_on_sc=False),
    )
    def k(table_hbm, ids_hbm, out_hbm, idx_v, rows_v, sem):
        wid = lax.axis_index("s") * _NC + lax.axis_index("c")

        def body(ch, carry):
            pltpu.sync_copy(ids_hbm.at[wid, ch], idx_v)
            descs = []
            for j in range(_DMAS_PER_CHUNK):
                descs.append(pltpu.async_copy(
                    table_hbm.at[idx_v.at[j]], rows_v.at[j], sem))
            for d in descs:
                d.wait()
            pltpu.sync_copy(rows_v, out_hbm.at[wid, ch])
            return carry

        lax.fori_loop(0, _NCH, body, 0)

    return k(table, ids4d)


def _highway(h, lo):
    gate = 1.0 / (1.0 + jnp.exp(-h[:, lo:lo + _DIM]))
    lin = h[:, lo + _DIM:lo + 2 * _DIM]
    nonlin = jnp.maximum(h[:, lo + 2 * _DIM:lo + 3 * _DIM], 0.0)
    return gate * (nonlin - lin) + lin


def _tc_highway(pre2, w2, b2):
    rows2 = _B // 2
    grid = (_L,)

    def body(x_ref, w_ref, b_ref, o_ref):
        x2 = x_ref[...].astype(jnp.bfloat16)
        h = jnp.dot(x2, w_ref[...], preferred_element_type=jnp.float32)
        h = h + b_ref[...]
        o_even = _highway(h, 0)
        o_odd = _highway(h, 3 * _DIM)
        st = jnp.stack([o_even, o_odd], axis=1)
        o_ref[...] = st.reshape(1, _B, _DIM)

    return pl.pallas_call(
        body,
        grid=grid,
        in_specs=[
            pl.BlockSpec((rows2, 2 * _LDIM), lambda i: (i, 0)),
            pl.BlockSpec((2 * _LDIM, 6 * _DIM), lambda i: (0, 0)),
            pl.BlockSpec((1, 6 * _DIM), lambda i: (0, 0)),
        ],
        out_specs=pl.BlockSpec((1, _B, _DIM), lambda i: (i, 0, 0)),
        out_shape=jax.ShapeDtypeStruct((_L, _B, _DIM), jnp.float32),
    )(pre2, w2, b2)


def kernel(sent_ids, learn_embed, gate_W, gate_b, lin_W, lin_b, nonlin_W, nonlin_b):
    ids = sent_ids.T.reshape(_NW, _NCH, _DMAS_PER_CHUNK, _IDX_MINOR).astype(jnp.int32)
    pre2 = _sc_gather(learn_embed, ids).reshape(_N // 2, 2 * _LDIM)
    w_cat = jnp.concatenate([gate_W, lin_W, nonlin_W], axis=1)
    zeros = jnp.zeros_like(w_cat)
    w2 = jnp.concatenate([
        jnp.concatenate([w_cat, zeros], axis=1),
        jnp.concatenate([zeros, w_cat], axis=1),
    ], axis=0).astype(jnp.bfloat16)
    b_cat = jnp.concatenate([gate_b - 2.0, lin_b, nonlin_b])
    b2 = jnp.concatenate([b_cat, b_cat]).reshape(1, 6 * _DIM)
    out_t = _tc_highway(pre2, w2, b2)
    return jnp.transpose(out_t, (1, 0, 2))


# speedup vs baseline: 2.6708x; 1.2001x over previous
import functools

import jax
import jax.numpy as jnp
from jax import lax
from jax.experimental import pallas as pl
from jax.experimental.pallas import tpu as pltpu
from jax.experimental.pallas import tpu_sc as plsc

_B, _L, _V, _LDIM, _DIM = 16384, 50, 1000000, 64, 128
_N = _B * _L

_NC, _NS = 2, 16
_NW = _NC * _NS
_ROWS_PER_W = _N // _NW
_IDX_MINOR = 128
_DMAS_PER_CHUNK = 8
_CHUNK = _DMAS_PER_CHUNK * _IDX_MINOR
_NCH = _ROWS_PER_W // _CHUNK


def _sc_gather(table, ids4d):
    mesh = plsc.VectorSubcoreMesh(core_axis_name="c", subcore_axis_name="s")

    @functools.partial(
        pl.kernel,
        mesh=mesh,
        out_type=jax.ShapeDtypeStruct(
            (_NW, _NCH, _DMAS_PER_CHUNK, _IDX_MINOR, _LDIM), jnp.float32),
        scratch_types=[
            pltpu.VMEM((_DMAS_PER_CHUNK, _IDX_MINOR), jnp.int32),
            pltpu.VMEM((_DMAS_PER_CHUNK, _IDX_MINOR, _LDIM), jnp.float32),
            pltpu.SemaphoreType.DMA,
        ],
        compiler_params=pltpu.CompilerParams(use_tc_tiling_on_sc=False),
    )
    def k(table_hbm, ids_hbm, out_hbm, idx_v, rows_v, sem):
        wid = lax.axis_index("s") * _NC + lax.axis_index("c")

        def body(ch, carry):
            pltpu.sync_copy(ids_hbm.at[wid, ch], idx_v)
            descs = []
            for j in range(_DMAS_PER_CHUNK):
                descs.append(pltpu.async_copy(
                    table_hbm.at[idx_v.at[j]], rows_v.at[j], sem))
            for d in descs:
                d.wait()
            pltpu.sync_copy(rows_v, out_hbm.at[wid, ch])
            return carry

        lax.fori_loop(0, _NCH, body, 0)

    return k(table, ids4d)


def _highway(h, lo):
    gate = 1.0 / (1.0 + jnp.exp(-h[:, lo:lo + _DIM]))
    lin = h[:, lo + _DIM:lo + 2 * _DIM]
    nonlin = jnp.maximum(h[:, lo + 2 * _DIM:lo + 3 * _DIM], 0.0)
    return gate * (nonlin - lin) + lin


def _tc_highway(pre2, w2, b2):
    rows2 = _B // 2
    grid = (_L,)

    def body(x_ref, w_ref, b_ref, o_ref):
        x2 = x_ref[...].astype(jnp.bfloat16)
        h = jnp.dot(x2, w_ref[...], preferred_element_type=jnp.float32)
        h = h + b_ref[...]
        o_ref[0, :rows2, :] = _highway(h, 0)
        o_ref[0, rows2:, :] = _highway(h, 3 * _DIM)

    return pl.pallas_call(
        body,
        grid=grid,
        in_specs=[
            pl.BlockSpec((rows2, 2 * _LDIM), lambda i: (i, 0)),
            pl.BlockSpec((2 * _LDIM, 6 * _DIM), lambda i: (0, 0)),
            pl.BlockSpec((1, 6 * _DIM), lambda i: (0, 0)),
        ],
        out_specs=pl.BlockSpec((1, _B, _DIM), lambda i: (i, 0, 0)),
        out_shape=jax.ShapeDtypeStruct((_L, _B, _DIM), jnp.float32),
    )(pre2, w2, b2)


def kernel(sent_ids, learn_embed, gate_W, gate_b, lin_W, lin_b, nonlin_W, nonlin_b):
    ids_wm = sent_ids.T.reshape(_L, 2, _B // 2).transpose(0, 2, 1)
    table_lin = jnp.pad(learn_embed, ((0, 0), (0, 2 * _LDIM - _LDIM))).reshape(
        2 * _V, _LDIM)
    ids = (2 * ids_wm.astype(jnp.int32)).reshape(
        _NW, _NCH, _DMAS_PER_CHUNK, _IDX_MINOR)
    pre2 = _sc_gather(table_lin, ids).reshape(_N // 2, 2 * _LDIM)
    w_cat = jnp.concatenate([gate_W, lin_W, nonlin_W], axis=1)
    zeros = jnp.zeros_like(w_cat)
    w2 = jnp.concatenate([
        jnp.concatenate([w_cat, zeros], axis=1),
        jnp.concatenate([zeros, w_cat], axis=1),
    ], axis=0).astype(jnp.bfloat16)
    b_cat = jnp.concatenate([gate_b - 2.0, lin_b, nonlin_b])
    b2 = jnp.concatenate([b_cat, b_cat]).reshape(1, 6 * _DIM)
    out_t = _tc_highway(pre2, w2, b2)
    return jnp.transpose(out_t, (1, 0, 2))


# speedup vs baseline: 3.1351x; 1.1739x over previous
import functools

import jax
import jax.numpy as jnp
from jax import lax
from jax.experimental import pallas as pl
from jax.experimental.pallas import tpu as pltpu
from jax.experimental.pallas import tpu_sc as plsc

_B, _L, _V, _LDIM, _DIM = 16384, 50, 1000000, 64, 128
_N = _B * _L

_NC, _NS = 2, 16
_NW = _NC * _NS
_ROWS_PER_W = _N // _NW
_IDX_MINOR = 128
_DMAS_PER_CHUNK = 8
_CHUNK = _DMAS_PER_CHUNK * _IDX_MINOR
_NCH = _ROWS_PER_W // _CHUNK


_HALF = _CHUNK // 2


def _sc_gather(table_lin, ids_nat):
    mesh = plsc.VectorSubcoreMesh(core_axis_name="c", subcore_axis_name="s")

    @functools.partial(
        pl.kernel,
        mesh=mesh,
        out_type=jax.ShapeDtypeStruct(
            (_NW, _NCH, _DMAS_PER_CHUNK, _IDX_MINOR, _LDIM), jnp.float32),
        scratch_types=[
            pltpu.VMEM((_CHUNK + 16,), jnp.int32),
            pltpu.VMEM((_CHUNK,), jnp.int32),
            pltpu.VMEM((_DMAS_PER_CHUNK, _IDX_MINOR, _LDIM), jnp.float32),
            pltpu.SemaphoreType.DMA,
            pltpu.SemaphoreType.DMA,
        ],
        compiler_params=pltpu.CompilerParams(use_tc_tiling_on_sc=False),
    )
    def k(table_hbm, ids_hbm, out_hbm, ab_v, idx_v, rows_v, sem_i, sem_g):
        wid = lax.axis_index("s") * _NC + lax.axis_index("c")

        def body(ch, carry):
            c = wid * _NCH + ch
            w = c // (_B // _CHUNK)
            mc = (c % (_B // _CHUNK)) * _HALF
            base_a = w * _B + mc
            da = pltpu.async_copy(
                ids_hbm.at[pl.ds(base_a, _HALF)], ab_v.at[pl.ds(0, _HALF)],
                sem_i)
            db = pltpu.async_copy(
                ids_hbm.at[pl.ds(base_a + _B // 2, _HALF)],
                ab_v.at[pl.ds(_HALF, _HALF)], sem_i)
            da.wait()
            db.wait()

            def ileave(q, carry2):
                ln = lax.iota(jnp.int32, 16)
                half = ln >> 1
                va = ab_v[pl.ds(8 * q, 16)]
                vb = ab_v[pl.ds(_HALF + 8 * q, 16)]
                pa = va.at[half].get(mode="promise_in_bounds")
                pb = vb.at[half].get(mode="promise_in_bounds")
                v = jnp.where((ln & 1) == 0, pa, pb)
                idx_v[pl.ds(16 * q, 16)] = v * 2
                return carry2

            lax.fori_loop(0, _CHUNK // 16, ileave, 0)

            descs = []
            for j in range(_DMAS_PER_CHUNK):
                descs.append(pltpu.async_copy(
                    table_hbm.at[idx_v.at[pl.ds(j * _IDX_MINOR, _IDX_MINOR)]],
                    rows_v.at[j], sem_g))
            for d in descs:
                d.wait()
            pltpu.sync_copy(rows_v, out_hbm.at[wid, ch])
            return carry

        lax.fori_loop(0, _NCH, body, 0)

    return k(table_lin, ids_nat)


def _highway(h, lo):
    gate = 1.0 / (1.0 + jnp.exp(-h[:, lo:lo + _DIM]))
    lin = h[:, lo + _DIM:lo + 2 * _DIM]
    nonlin = jnp.maximum(h[:, lo + 2 * _DIM:lo + 3 * _DIM], 0.0)
    return gate * (nonlin - lin) + lin


def _tc_highway(pre2, w2, b2):
    rows2 = _B // 2
    grid = (_L,)

    def body(x_ref, w_ref, b_ref, o_ref):
        x2 = x_ref[...].astype(jnp.bfloat16)
        h = jnp.dot(x2, w_ref[...], preferred_element_type=jnp.float32)
        h = h + b_ref[...]
        o_ref[0, :rows2, :] = _highway(h, 0)
        o_ref[0, rows2:, :] = _highway(h, 3 * _DIM)

    return pl.pallas_call(
        body,
        grid=grid,
        in_specs=[
            pl.BlockSpec((rows2, 2 * _LDIM), lambda i: (i, 0)),
            pl.BlockSpec((2 * _LDIM, 6 * _DIM), lambda i: (0, 0)),
            pl.BlockSpec((1, 6 * _DIM), lambda i: (0, 0)),
        ],
        out_specs=pl.BlockSpec((1, _B, _DIM), lambda i: (i, 0, 0)),
        out_shape=jax.ShapeDtypeStruct((_L, _B, _DIM), jnp.float32),
    )(pre2, w2, b2)


def kernel(sent_ids, learn_embed, gate_W, gate_b, lin_W, lin_b, nonlin_W, nonlin_b):
    table_lin = jnp.pad(learn_embed, ((0, 0), (0, 2 * _LDIM - _LDIM))).reshape(
        2 * _V, _LDIM)
    ids_nat = sent_ids.T.reshape(_N).astype(jnp.int32)
    pre2 = _sc_gather(table_lin, ids_nat).reshape(_N // 2, 2 * _LDIM)
    w_cat = jnp.concatenate([gate_W, lin_W, nonlin_W], axis=1)
    zeros = jnp.zeros_like(w_cat)
    w2 = jnp.concatenate([
        jnp.concatenate([w_cat, zeros], axis=1),
        jnp.concatenate([zeros, w_cat], axis=1),
    ], axis=0).astype(jnp.bfloat16)
    b_cat = jnp.concatenate([gate_b - 2.0, lin_b, nonlin_b])
    b2 = jnp.concatenate([b_cat, b_cat]).reshape(1, 6 * _DIM)
    out_t = _tc_highway(pre2, w2, b2)
    return jnp.transpose(out_t, (1, 0, 2))


# speedup vs baseline: 3.1366x; 1.0005x over previous
import functools

import jax
import jax.numpy as jnp
from jax import lax
from jax.experimental import pallas as pl
from jax.experimental.pallas import tpu as pltpu
from jax.experimental.pallas import tpu_sc as plsc

_B, _L, _V, _LDIM, _DIM = 16384, 50, 1000000, 64, 128
_N = _B * _L

_NC, _NS = 2, 16
_NW = _NC * _NS
_ROWS_PER_W = _N // _NW
_IDX_MINOR = 128
_DMAS_PER_CHUNK = 8
_CHUNK = _DMAS_PER_CHUNK * _IDX_MINOR
_NCH = _ROWS_PER_W // _CHUNK


_HALF = _CHUNK // 2


def _sc_gather(table_lin, ids_nat):
    mesh = plsc.VectorSubcoreMesh(core_axis_name="c", subcore_axis_name="s")

    @functools.partial(
        pl.kernel,
        mesh=mesh,
        out_type=jax.ShapeDtypeStruct(
            (_NW, _NCH, _DMAS_PER_CHUNK, _IDX_MINOR, _LDIM), jnp.float32),
        scratch_types=[
            pltpu.VMEM((_CHUNK + 16,), jnp.int32),
            pltpu.VMEM((_CHUNK,), jnp.int32),
            pltpu.VMEM((_DMAS_PER_CHUNK, _IDX_MINOR, _LDIM), jnp.float32),
            pltpu.SemaphoreType.DMA,
            pltpu.SemaphoreType.DMA,
        ],
        compiler_params=pltpu.CompilerParams(use_tc_tiling_on_sc=False),
    )
    def k(table_hbm, ids_hbm, out_hbm, ab_v, idx_v, rows_v, sem_i, sem_g):
        wid = lax.axis_index("s") * _NC + lax.axis_index("c")

        def body(ch, carry):
            c = wid * _NCH + ch
            w = c // (_B // _CHUNK)
            mc = (c % (_B // _CHUNK)) * _HALF
            base_a = w * _B + mc
            da = pltpu.async_copy(
                ids_hbm.at[pl.ds(base_a, _HALF)], ab_v.at[pl.ds(0, _HALF)],
                sem_i)
            db = pltpu.async_copy(
                ids_hbm.at[pl.ds(base_a + _B // 2, _HALF)],
                ab_v.at[pl.ds(_HALF, _HALF)], sem_i)
            da.wait()
            db.wait()

            def ileave(q, carry2):
                ln = lax.iota(jnp.int32, 16)
                half = ln >> 1
                va = ab_v[pl.ds(8 * q, 16)]
                vb = ab_v[pl.ds(_HALF + 8 * q, 16)]
                pa = va.at[half].get(mode="promise_in_bounds")
                pb = vb.at[half].get(mode="promise_in_bounds")
                v = jnp.where((ln & 1) == 0, pa, pb)
                idx_v[pl.ds(16 * q, 16)] = v * 2
                return carry2

            lax.fori_loop(0, _CHUNK // 16, ileave, 0)

            descs = []
            for j in range(_DMAS_PER_CHUNK):
                descs.append(pltpu.async_copy(
                    table_hbm.at[idx_v.at[pl.ds(j * _IDX_MINOR, _IDX_MINOR)]],
                    rows_v.at[j], sem_g))
            for d in descs:
                d.wait()
            pltpu.sync_copy(rows_v, out_hbm.at[wid, ch])
            return carry

        lax.fori_loop(0, _NCH, body, 0)

    return k(table_lin, ids_nat)


def _highway(h, lo):
    gate = 1.0 / (1.0 + jnp.exp(-h[:, lo:lo + _DIM]))
    lin = h[:, lo + _DIM:lo + 2 * _DIM]
    nonlin = jnp.maximum(h[:, lo + 2 * _DIM:lo + 3 * _DIM], 0.0)
    return gate * (nonlin - lin) + lin


def _tc_highway(pre2, w2, b2):
    rows2 = _B // 2
    grid = (_L,)

    def body(x_ref, w_ref, b_ref, o_ref):
        x2 = x_ref[...].astype(jnp.bfloat16)
        h = jnp.dot(x2, w_ref[...], preferred_element_type=jnp.float32)
        h = h + b_ref[...]
        o_ref[0, :rows2, :] = _highway(h, 0)
        o_ref[0, rows2:, :] = _highway(h, 3 * _DIM)

    return pl.pallas_call(
        body,
        grid=grid,
        in_specs=[
            pl.BlockSpec((rows2, 2 * _LDIM), lambda i: (i, 0)),
            pl.BlockSpec((2 * _LDIM, 6 * _DIM), lambda i: (0, 0)),
            pl.BlockSpec((1, 6 * _DIM), lambda i: (0, 0)),
        ],
        out_specs=pl.BlockSpec((1, _B, _DIM), lambda i: (i, 0, 0)),
        out_shape=jax.ShapeDtypeStruct((_L, _B, _DIM), jnp.float32),
    )(pre2, w2, b2)


def kernel(sent_ids, learn_embed, gate_W, gate_b, lin_W, lin_b, nonlin_W, nonlin_b):
    table_lin = jnp.concatenate(
        [learn_embed, jnp.zeros_like(learn_embed)], axis=1).reshape(
        2 * _V, _LDIM)
    ids_nat = sent_ids.T.reshape(_N).astype(jnp.int32)
    pre2 = _sc_gather(table_lin, ids_nat).reshape(_N // 2, 2 * _LDIM)
    w_cat = jnp.concatenate([gate_W, lin_W, nonlin_W], axis=1)
    zeros = jnp.zeros_like(w_cat)
    w2 = jnp.concatenate([
        jnp.concatenate([w_cat, zeros], axis=1),
        jnp.concatenate([zeros, w_cat], axis=1),
    ], axis=0).astype(jnp.bfloat16)
    b_cat = jnp.concatenate([gate_b - 2.0, lin_b, nonlin_b])
    b2 = jnp.concatenate([b_cat, b_cat]).reshape(1, 6 * _DIM)
    out_t = _tc_highway(pre2, w2, b2)
    return jnp.transpose(out_t, (1, 0, 2))


# speedup vs baseline: 3.2893x; 1.0487x over previous
import functools

import jax
import jax.numpy as jnp
from jax import lax
from jax.experimental import pallas as pl
from jax.experimental.pallas import tpu as pltpu
from jax.experimental.pallas import tpu_sc as plsc

_B, _L, _V, _LDIM, _DIM = 16384, 50, 1000000, 64, 128
_N = _B * _L

_NC, _NS = 2, 16
_NW = _NC * _NS
_ROWS_PER_W = _N // _NW
_IDX_MINOR = 128
_DMAS_PER_CHUNK = 8
_CHUNK = _DMAS_PER_CHUNK * _IDX_MINOR
_NCH = _ROWS_PER_W // _CHUNK


_HALF = _CHUNK // 2


def _sc_gather(table_lin, ids_nat):
    mesh = plsc.VectorSubcoreMesh(core_axis_name="c", subcore_axis_name="s")

    @functools.partial(
        pl.kernel,
        mesh=mesh,
        out_type=jax.ShapeDtypeStruct(
            (_NW, _NCH, _DMAS_PER_CHUNK, _IDX_MINOR, _LDIM), jnp.float32),
        scratch_types=[
            pltpu.VMEM((_CHUNK + 16,), jnp.int32),
            pltpu.VMEM((_CHUNK,), jnp.int32),
            pltpu.VMEM((_DMAS_PER_CHUNK, _IDX_MINOR, _LDIM), jnp.float32),
            pltpu.SemaphoreType.DMA,
            pltpu.SemaphoreType.DMA,
        ],
        compiler_params=pltpu.CompilerParams(use_tc_tiling_on_sc=False),
    )
    def k(table_hbm, ids_hbm, out_hbm, ab_v, idx_v, rows_v, sem_i, sem_g):
        wid = lax.axis_index("s") * _NC + lax.axis_index("c")

        def body(ch, carry):
            c = wid * _NCH + ch
            w = c // (_B // _CHUNK)
            mc = (c % (_B // _CHUNK)) * _HALF
            base_a = w * _B + mc
            da = pltpu.async_copy(
                ids_hbm.at[pl.ds(base_a, _HALF)], ab_v.at[pl.ds(0, _HALF)],
                sem_i)
            db = pltpu.async_copy(
                ids_hbm.at[pl.ds(base_a + _B // 2, _HALF)],
                ab_v.at[pl.ds(_HALF, _HALF)], sem_i)
            da.wait()
            db.wait()

            def ileave(q, carry2):
                ln = lax.iota(jnp.int32, 16)
                half = ln >> 1
                va = ab_v[pl.ds(8 * q, 16)]
                vb = ab_v[pl.ds(_HALF + 8 * q, 16)]
                pa = va.at[half].get(mode="promise_in_bounds")
                pb = vb.at[half].get(mode="promise_in_bounds")
                v = jnp.where((ln & 1) == 0, pa, pb)
                idx_v[pl.ds(16 * q, 16)] = v * 2
                return carry2

            lax.fori_loop(0, _CHUNK // 16, ileave, 0)

            descs = []
            for j in range(_DMAS_PER_CHUNK):
                descs.append(pltpu.async_copy(
                    table_hbm.at[idx_v.at[pl.ds(j * _IDX_MINOR, _IDX_MINOR)]],
                    rows_v.at[j], sem_g))
            for d in descs:
                d.wait()
            pltpu.sync_copy(rows_v, out_hbm.at[wid, ch])
            return carry

        lax.fori_loop(0, _NCH, body, 0)

    return k(table_lin, ids_nat)


def _tc_tablepad(table_t, eye):
    cols = 2048
    grid = (_V // cols,)

    def body(x_ref, e_ref, o_ref):
        xt = lax.dot_general(x_ref[...], e_ref[...],
                             (((0,), (0,)), ((), ())),
                             preferred_element_type=jnp.float32)
        o_ref[...] = jnp.concatenate(
            [xt, jnp.zeros((cols, _LDIM), jnp.float32)], axis=1)

    return pl.pallas_call(
        body,
        grid=grid,
        in_specs=[
            pl.BlockSpec((_LDIM, cols), lambda i: (0, i)),
            pl.BlockSpec((_LDIM, _LDIM), lambda i: (0, 0)),
        ],
        out_specs=pl.BlockSpec((cols, 2 * _LDIM), lambda i: (i, 0)),
        out_shape=jax.ShapeDtypeStruct((_V, 2 * _LDIM), jnp.float32),
    )(table_t, eye)


def _highway(h, lo):
    gate = 1.0 / (1.0 + jnp.exp(-h[:, lo:lo + _DIM]))
    lin = h[:, lo + _DIM:lo + 2 * _DIM]
    nonlin = jnp.maximum(h[:, lo + 2 * _DIM:lo + 3 * _DIM], 0.0)
    return gate * (nonlin - lin) + lin


def _tc_highway(pre2, w2, b2):
    rows2 = _B // 2
    grid = (_L,)

    def body(x_ref, w_ref, b_ref, o_ref):
        x2 = x_ref[...].astype(jnp.bfloat16)
        h = jnp.dot(x2, w_ref[...], preferred_element_type=jnp.float32)
        h = h + b_ref[...]
        o_ref[0, :rows2, :] = _highway(h, 0)
        o_ref[0, rows2:, :] = _highway(h, 3 * _DIM)

    return pl.pallas_call(
        body,
        grid=grid,
        in_specs=[
            pl.BlockSpec((rows2, 2 * _LDIM), lambda i: (i, 0)),
            pl.BlockSpec((2 * _LDIM, 6 * _DIM), lambda i: (0, 0)),
            pl.BlockSpec((1, 6 * _DIM), lambda i: (0, 0)),
        ],
        out_specs=pl.BlockSpec((1, _B, _DIM), lambda i: (i, 0, 0)),
        out_shape=jax.ShapeDtypeStruct((_L, _B, _DIM), jnp.float32),
    )(pre2, w2, b2)


def kernel(sent_ids, learn_embed, gate_W, gate_b, lin_W, lin_b, nonlin_W, nonlin_b):
    table_lin = _tc_tablepad(
        learn_embed.T, jnp.eye(_LDIM, dtype=jnp.float32)).reshape(
        2 * _V, _LDIM)
    ids_nat = sent_ids.T.reshape(_N).astype(jnp.int32)
    pre2 = _sc_gather(table_lin, ids_nat).reshape(_N // 2, 2 * _LDIM)
    w_cat = jnp.concatenate([gate_W, lin_W, nonlin_W], axis=1)
    zeros = jnp.zeros_like(w_cat)
    w2 = jnp.concatenate([
        jnp.concatenate([w_cat, zeros], axis=1),
        jnp.concatenate([zeros, w_cat], axis=1),
    ], axis=0).astype(jnp.bfloat16)
    b_cat = jnp.concatenate([gate_b - 2.0, lin_b, nonlin_b])
    b2 = jnp.concatenate([b_cat, b_cat]).reshape(1, 6 * _DIM)
    out_t = _tc_highway(pre2, w2, b2)
    return jnp.transpose(out_t, (1, 0, 2))


# speedup vs baseline: 4.3290x; 1.3161x over previous
import functools

import jax
import jax.numpy as jnp
from jax import lax
from jax.experimental import pallas as pl
from jax.experimental.pallas import tpu as pltpu
from jax.experimental.pallas import tpu_sc as plsc

_B, _L, _V, _LDIM, _DIM = 16384, 50, 1000000, 64, 128
_N = _B * _L

_NC, _NS = 2, 16
_NW = _NC * _NS
_ROWS_PER_W = _N // _NW
_IDX_MINOR = 128
_DMAS_PER_CHUNK = 8
_CHUNK = _DMAS_PER_CHUNK * _IDX_MINOR
_NCH = _ROWS_PER_W // _CHUNK


_HALF = _CHUNK // 2


def _sc_gather(table_lin, ids_nat):
    mesh = plsc.VectorSubcoreMesh(core_axis_name="c", subcore_axis_name="s")

    @functools.partial(
        pl.kernel,
        mesh=mesh,
        out_type=jax.ShapeDtypeStruct(
            (_NW, _NCH, _DMAS_PER_CHUNK, _IDX_MINOR, _LDIM), jnp.float32),
        scratch_types=[
            pltpu.VMEM((_CHUNK + 16,), jnp.int32),
            pltpu.VMEM((_CHUNK,), jnp.int32),
            pltpu.VMEM((_DMAS_PER_CHUNK, _IDX_MINOR, _LDIM), jnp.float32),
            pltpu.SemaphoreType.DMA,
            pltpu.SemaphoreType.DMA,
        ],
        compiler_params=pltpu.CompilerParams(use_tc_tiling_on_sc=False),
    )
    def k(table_hbm, ids_hbm, out_hbm, ab_v, idx_v, rows_v, sem_i, sem_g):
        wid = lax.axis_index("s") * _NC + lax.axis_index("c")

        def body(ch, carry):
            c = wid * _NCH + ch
            w = c // (_B // _CHUNK)
            mc = (c % (_B // _CHUNK)) * _HALF
            base_a = w * _B + mc
            da = pltpu.async_copy(
                ids_hbm.at[pl.ds(base_a, _HALF)], ab_v.at[pl.ds(0, _HALF)],
                sem_i)
            db = pltpu.async_copy(
                ids_hbm.at[pl.ds(base_a + _B // 2, _HALF)],
                ab_v.at[pl.ds(_HALF, _HALF)], sem_i)
            da.wait()
            db.wait()

            def ileave(q, carry2):
                ln = lax.iota(jnp.int32, 16)
                half = ln >> 1
                va = ab_v[pl.ds(8 * q, 16)]
                vb = ab_v[pl.ds(_HALF + 8 * q, 16)]
                pa = va.at[half].get(mode="promise_in_bounds")
                pb = vb.at[half].get(mode="promise_in_bounds")
                v = jnp.where((ln & 1) == 0, pa, pb)
                idx_v[pl.ds(16 * q, 16)] = v * 2
                return carry2

            lax.fori_loop(0, _CHUNK // 16, ileave, 0)

            descs = []
            for j in range(_DMAS_PER_CHUNK):
                descs.append(pltpu.async_copy(
                    table_hbm.at[idx_v.at[pl.ds(j * _IDX_MINOR, _IDX_MINOR)]],
                    rows_v.at[j], sem_g))
            for d in descs:
                d.wait()
            pltpu.sync_copy(rows_v, out_hbm.at[wid, ch])
            return carry

        lax.fori_loop(0, _NCH, body, 0)

    return k(table_lin, ids_nat)


def _tc_tablepad(table_t, eye):
    cols = 8192
    grid = (_V // cols,)

    def body(x_ref, e_ref, o_ref):
        xt = lax.dot_general(x_ref[...], e_ref[...],
                             (((0,), (0,)), ((), ())),
                             preferred_element_type=jnp.float32)
        o_ref[...] = jnp.concatenate(
            [xt, jnp.zeros((cols, _LDIM), jnp.float32)], axis=1)

    return pl.pallas_call(
        body,
        grid=grid,
        in_specs=[
            pl.BlockSpec((_LDIM, cols), lambda i: (0, i)),
            pl.BlockSpec((_LDIM, _LDIM), lambda i: (0, 0)),
        ],
        out_specs=pl.BlockSpec((cols, 2 * _LDIM), lambda i: (i, 0)),
        out_shape=jax.ShapeDtypeStruct((_V, 2 * _LDIM), jnp.float32),
    )(table_t, eye)


def _highway(h, lo):
    gate = 1.0 / (1.0 + jnp.exp(-h[:, lo:lo + _DIM]))
    lin = h[:, lo + _DIM:lo + 2 * _DIM]
    nonlin = jnp.maximum(h[:, lo + 2 * _DIM:lo + 3 * _DIM], 0.0)
    return gate * (nonlin - lin) + lin


def _tc_highway(pre2, w2, b2):
    rows2 = _B // 2
    grid = (_L,)

    def body(x_ref, w_ref, b_ref, o_ref):
        x2 = x_ref[...].astype(jnp.bfloat16)
        h = jnp.dot(x2, w_ref[...], preferred_element_type=jnp.float32)
        h = h + b_ref[...]
        o_ref[0, :rows2, :] = _highway(h, 0)
        o_ref[0, rows2:, :] = _highway(h, 3 * _DIM)

    return pl.pallas_call(
        body,
        grid=grid,
        in_specs=[
            pl.BlockSpec((rows2, 2 * _LDIM), lambda i: (i, 0)),
            pl.BlockSpec((2 * _LDIM, 6 * _DIM), lambda i: (0, 0)),
            pl.BlockSpec((1, 6 * _DIM), lambda i: (0, 0)),
        ],
        out_specs=pl.BlockSpec((1, _B, _DIM), lambda i: (i, 0, 0)),
        out_shape=jax.ShapeDtypeStruct((_L, _B, _DIM), jnp.float32),
    )(pre2, w2, b2)


def kernel(sent_ids, learn_embed, gate_W, gate_b, lin_W, lin_b, nonlin_W, nonlin_b):
    table_lin = _tc_tablepad(
        learn_embed.T, jnp.eye(_LDIM, dtype=jnp.float32)).reshape(
        2 * _V, _LDIM)
    ids_nat = sent_ids.T.reshape(_N).astype(jnp.int32)
    pre2 = _sc_gather(table_lin, ids_nat).reshape(_N // 2, 2 * _LDIM)
    w_cat = jnp.concatenate([gate_W, lin_W, nonlin_W], axis=1)
    zeros = jnp.zeros_like(w_cat)
    w2 = jnp.concatenate([
        jnp.concatenate([w_cat, zeros], axis=1),
        jnp.concatenate([zeros, w_cat], axis=1),
    ], axis=0).astype(jnp.bfloat16)
    b_cat = jnp.concatenate([gate_b - 2.0, lin_b, nonlin_b])
    b2 = jnp.concatenate([b_cat, b_cat]).reshape(1, 6 * _DIM)
    out_t = _tc_highway(pre2, w2, b2)
    return jnp.transpose(out_t, (1, 0, 2))


# speedup vs baseline: 4.3316x; 1.0006x over previous
import functools

import jax
import jax.numpy as jnp
from jax import lax
from jax.experimental import pallas as pl
from jax.experimental.pallas import tpu as pltpu
from jax.experimental.pallas import tpu_sc as plsc

_B, _L, _V, _LDIM, _DIM = 16384, 50, 1000000, 64, 128
_N = _B * _L

_NC, _NS = 2, 16
_NW = _NC * _NS
_ROWS_PER_W = _N // _NW
_IDX_MINOR = 128
_DMAS_PER_CHUNK = 8
_CHUNK = _DMAS_PER_CHUNK * _IDX_MINOR
_NCH = _ROWS_PER_W // _CHUNK


_HALF = _CHUNK // 2


def _sc_gather(table_lin, ids_nat):
    mesh = plsc.VectorSubcoreMesh(core_axis_name="c", subcore_axis_name="s")

    @functools.partial(
        pl.kernel,
        mesh=mesh,
        out_type=jax.ShapeDtypeStruct(
            (_NW, _NCH, _DMAS_PER_CHUNK, _IDX_MINOR, _LDIM), jnp.float32),
        scratch_types=[
            pltpu.VMEM((_CHUNK + 16,), jnp.int32),
            pltpu.VMEM((_CHUNK,), jnp.int32),
            pltpu.VMEM((_DMAS_PER_CHUNK, _IDX_MINOR, _LDIM), jnp.float32),
            pltpu.SemaphoreType.DMA,
            pltpu.SemaphoreType.DMA,
        ],
        compiler_params=pltpu.CompilerParams(use_tc_tiling_on_sc=False),
    )
    def k(table_hbm, ids_hbm, out_hbm, ab_v, idx_v, rows_v, sem_i, sem_g):
        wid = lax.axis_index("s") * _NC + lax.axis_index("c")

        def body(ch, carry):
            c = wid * _NCH + ch
            w = c // (_B // _CHUNK)
            mc = (c % (_B // _CHUNK)) * _HALF
            base_a = w * _B + mc
            da = pltpu.async_copy(
                ids_hbm.at[pl.ds(base_a, _HALF)], ab_v.at[pl.ds(0, _HALF)],
                sem_i)
            db = pltpu.async_copy(
                ids_hbm.at[pl.ds(base_a + _B // 2, _HALF)],
                ab_v.at[pl.ds(_HALF, _HALF)], sem_i)
            da.wait()
            db.wait()

            def ileave(q, carry2):
                ln = lax.iota(jnp.int32, 16)
                half = ln >> 1
                va = ab_v[pl.ds(8 * q, 16)]
                vb = ab_v[pl.ds(_HALF + 8 * q, 16)]
                pa = va.at[half].get(mode="promise_in_bounds")
                pb = vb.at[half].get(mode="promise_in_bounds")
                v = jnp.where((ln & 1) == 0, pa, pb)
                idx_v[pl.ds(16 * q, 16)] = v * 2
                return carry2

            lax.fori_loop(0, _CHUNK // 16, ileave, 0)

            descs = []
            for j in range(_DMAS_PER_CHUNK):
                descs.append(pltpu.async_copy(
                    table_hbm.at[idx_v.at[pl.ds(j * _IDX_MINOR, _IDX_MINOR)]],
                    rows_v.at[j], sem_g))
            for d in descs:
                d.wait()
            pltpu.sync_copy(rows_v, out_hbm.at[wid, ch])
            return carry

        lax.fori_loop(0, _NCH, body, 0)

    return k(table_lin, ids_nat)


def _tc_tablepad(table_t, eye):
    cols = 8192
    grid = (pl.cdiv(_V, cols),)

    def body(x_ref, e_ref, o_ref):
        xt = lax.dot_general(x_ref[...], e_ref[...],
                             (((0,), (0,)), ((), ())),
                             preferred_element_type=jnp.float32)
        o_ref[...] = jnp.concatenate(
            [xt, jnp.zeros((cols, _LDIM), jnp.float32)], axis=1)

    return pl.pallas_call(
        body,
        grid=grid,
        in_specs=[
            pl.BlockSpec((_LDIM, cols), lambda i: (0, i)),
            pl.BlockSpec((_LDIM, _LDIM), lambda i: (0, 0)),
        ],
        out_specs=pl.BlockSpec((cols, 2 * _LDIM), lambda i: (i, 0)),
        out_shape=jax.ShapeDtypeStruct((_V, 2 * _LDIM), jnp.float32),
    )(table_t, eye)


def _highway(h, lo):
    gate = 1.0 / (1.0 + jnp.exp(-h[:, lo:lo + _DIM]))
    lin = h[:, lo + _DIM:lo + 2 * _DIM]
    nonlin = jnp.maximum(h[:, lo + 2 * _DIM:lo + 3 * _DIM], 0.0)
    return gate * (nonlin - lin) + lin


def _tc_highway(pre2, w2, b2):
    rows2 = _B // 2
    grid = (_L,)

    def body(x_ref, w_ref, b_ref, o_ref):
        x2 = x_ref[...].astype(jnp.bfloat16)
        h = jnp.dot(x2, w_ref[...], preferred_element_type=jnp.float32)
        h = h + b_ref[...]
        o_ref[0, :rows2, :] = _highway(h, 0)
        o_ref[0, rows2:, :] = _highway(h, 3 * _DIM)

    return pl.pallas_call(
        body,
        grid=grid,
        in_specs=[
            pl.BlockSpec((rows2, 2 * _LDIM), lambda i: (i, 0)),
            pl.BlockSpec((2 * _LDIM, 6 * _DIM), lambda i: (0, 0)),
            pl.BlockSpec((1, 6 * _DIM), lambda i: (0, 0)),
        ],
        out_specs=pl.BlockSpec((1, _B, _DIM), lambda i: (i, 0, 0)),
        out_shape=jax.ShapeDtypeStruct((_L, _B, _DIM), jnp.float32),
    )(pre2, w2, b2)


def kernel(sent_ids, learn_embed, gate_W, gate_b, lin_W, lin_b, nonlin_W, nonlin_b):
    table_lin = _tc_tablepad(
        learn_embed.T, jnp.eye(_LDIM, dtype=jnp.float32)).reshape(
        2 * _V, _LDIM)
    ids_nat = sent_ids.T.reshape(_N).astype(jnp.int32)
    pre2 = _sc_gather(table_lin, ids_nat).reshape(_N // 2, 2 * _LDIM)
    w_cat = jnp.concatenate([gate_W, lin_W, nonlin_W], axis=1)
    zeros = jnp.zeros_like(w_cat)
    w2 = jnp.concatenate([
        jnp.concatenate([w_cat, zeros], axis=1),
        jnp.concatenate([zeros, w_cat], axis=1),
    ], axis=0).astype(jnp.bfloat16)
    b_cat = jnp.concatenate([gate_b - 2.0, lin_b, nonlin_b])
    b2 = jnp.concatenate([b_cat, b_cat]).reshape(1, 6 * _DIM)
    out_t = _tc_highway(pre2, w2, b2)
    return jnp.transpose(out_t, (1, 0, 2))
